# baseline (speedup 1.0000x reference)
import functools

import jax
import jax.numpy as jnp
from jax import lax
from jax.experimental import pallas as pl
from jax.experimental.pallas import tpu as pltpu
from jax.experimental.pallas import tpu_sc as plsc

N = 10000
E = 320000
D = 128
DOUT = 64

NC = 2
NS = 16
NW = NC * NS
EPW = E // NW
CH = 80
NCHUNK = EPW // CH
NP = 10240
RPT = NP // NS


def _sc_agg_body(h_hbm, src_hbm, dst2_hbm, zrow_hbm, out_hbm,
                 sidx, didx, rows, acc, gsem, ssem0, ssem1):
    ssems = (ssem0, ssem1)
    cid = lax.axis_index("c")
    sid = lax.axis_index("s")
    wid = cid * NS + sid
    r0 = sid * RPT

    pltpu.sync_copy(zrow_hbm, acc.at[pl.ds(r0, RPT)])
    pltpu.sync_copy(src_hbm.at[pl.ds(wid * EPW, EPW)], sidx)
    pltpu.sync_copy(dst2_hbm.at[wid], didx)
    plsc.subcore_barrier()

    pltpu.async_copy(h_hbm.at[sidx.at[pl.ds(0, CH)]], rows.at[0], gsem)

    def pair(j2, carry):
        for b in range(2):
            j = 2 * j2 + b
            pltpu.make_async_copy(h_hbm.at[pl.ds(0, CH)], rows.at[b], gsem).wait()
            pltpu.async_copy(rows.at[b], acc.at[didx.at[j]], ssems[b],
                             add=True)

            @pl.when(j > 0)
            def _():
                pltpu.make_async_copy(h_hbm.at[pl.ds(0, CH)], rows.at[1 - b],
                                      ssems[1 - b]).wait()

            @pl.when(j < NCHUNK - 1)
            def _():
                off = (j + 1) * CH
                pltpu.async_copy(h_hbm.at[sidx.at[pl.ds(off, CH)]],
                                 rows.at[1 - b], gsem)
        return carry

    lax.fori_loop(0, NCHUNK // 2, pair, 0)
    pltpu.make_async_copy(h_hbm.at[pl.ds(0, CH)], rows.at[0], gsem).wait()
    pltpu.async_copy(rows.at[0], acc.at[didx.at[NCHUNK - 1]], ssems[0],
                     add=True)
    pltpu.make_async_copy(h_hbm.at[pl.ds(0, CH)], rows.at[1], ssems[1]).wait()
    pltpu.make_async_copy(h_hbm.at[pl.ds(0, CH)], rows.at[0], ssems[0]).wait()
    plsc.subcore_barrier()

    pltpu.sync_copy(acc.at[pl.ds(r0, RPT)], out_hbm.at[cid, pl.ds(r0, RPT)])


_sc_agg = pl.kernel(
    _sc_agg_body,
    out_type=jax.ShapeDtypeStruct((NC, NP, D), jnp.float32),
    mesh=plsc.VectorSubcoreMesh(core_axis_name="c", subcore_axis_name="s"),
    scratch_types=[
        pltpu.VMEM((EPW,), jnp.int32),
        pltpu.VMEM((NCHUNK, CH), jnp.int32),
        pltpu.VMEM((2, CH, D), jnp.float32),
        pltpu.VMEM_SHARED((NP, D), jnp.float32),
        pltpu.SemaphoreType.DMA,
        pltpu.SemaphoreType.DMA,
        pltpu.SemaphoreType.DMA,
    ],
    name="sc_seg_agg",
)


def _sc_cnt_body(dst2_hbm, zrow_hbm, out_hbm, didx, rows1, acc):
    cid = lax.axis_index("c")
    sid = lax.axis_index("s")
    wid = cid * NS + sid
    r0 = sid * RPT

    pltpu.sync_copy(zrow_hbm, acc.at[pl.ds(r0, RPT)])
    pltpu.sync_copy(dst2_hbm.at[wid], didx)
    ones16 = jnp.ones((16,), jnp.float32)

    def fill(r, carry):
        for k in range(D // 16):
            rows1[r, pl.ds(16 * k, 16)] = ones16
        return carry

    lax.fori_loop(0, CH, fill, 0)
    plsc.subcore_barrier()

    def chunk(j, carry):
        pltpu.sync_copy(rows1, acc.at[didx.at[j]], add=True)
        return carry

    lax.fori_loop(0, NCHUNK, chunk, 0)
    plsc.subcore_barrier()

    pltpu.sync_copy(acc.at[pl.ds(r0, RPT)], out_hbm.at[cid, pl.ds(r0, RPT)])


_sc_cnt = pl.kernel(
    _sc_cnt_body,
    out_type=jax.ShapeDtypeStruct((NC, NP, D), jnp.float32),
    mesh=plsc.VectorSubcoreMesh(core_axis_name="c", subcore_axis_name="s"),
    scratch_types=[
        pltpu.VMEM((NCHUNK, CH), jnp.int32),
        pltpu.VMEM((CH, D), jnp.float32),
        pltpu.VMEM_SHARED((NP, D), jnp.float32),
    ],
    name="sc_seg_cnt",
)


_TCB = 1000


def _tc_layer_body(relu, part_ref, cnt_ref, h_ref, wl_ref, wr_ref, b_ref, o_ref):
    p = part_ref[...]
    c = cnt_ref[...]
    rc = 1.0 / jnp.maximum(c[:, 0:1] + c[:, 1:2], 1.0)
    mean = (p[0] + p[1]) * rc
    y = jnp.dot(mean, wl_ref[...], preferred_element_type=jnp.float32)
    y = y + jnp.dot(h_ref[...], wr_ref[...], preferred_element_type=jnp.float32)
    y = y + b_ref[...]
    if relu:
        y = jnp.maximum(y, 0.0)
    o_ref[...] = y


def _tc_layer(part, cnt2, hprev, wl_t, wr_t, b, relu):
    dout = wl_t.shape[1]
    grid = (N // _TCB,)
    return pl.pallas_call(
        functools.partial(_tc_layer_body, relu),
        grid=grid,
        in_specs=[
            pl.BlockSpec((2, _TCB, D), lambda i: (0, i, 0)),
            pl.BlockSpec((_TCB, 2), lambda i: (i, 0)),
            pl.BlockSpec((_TCB, D), lambda i: (i, 0)),
            pl.BlockSpec((D, dout), lambda i: (0, 0)),
            pl.BlockSpec((D, dout), lambda i: (0, 0)),
            pl.BlockSpec((1, dout), lambda i: (0, 0)),
        ],
        out_specs=pl.BlockSpec((_TCB, dout), lambda i: (i, 0)),
        out_shape=jax.ShapeDtypeStruct((N, dout), jnp.float32),
        name="tc_sage_linear",
    )(part, cnt2, hprev, wl_t, wr_t, b)


def kernel(x, edge_index, W1l, b1l, W1r, W2l, b2l, W2r,
           Wml, bml, Wmr, Wsl, bsl, Wsr):
    src = edge_index[0]
    dst = edge_index[1]
    zrow = jnp.zeros((RPT, D), jnp.float32)
    dst2 = dst.reshape(NW, NCHUNK, CH)
    cnt = _sc_cnt(dst2, zrow)
    cnt2 = jnp.concatenate([cnt[0, :, :1], cnt[1, :, :1]], axis=1)
    agg1 = _sc_agg(x, src, dst2, zrow)

    h1 = _tc_layer(agg1, cnt2, x, W1l.T, W1r.T, b1l.reshape(1, -1), True)
    agg2 = _sc_agg(h1, src, dst2, zrow)
    h2 = _tc_layer(agg2, cnt2, h1, W2l.T, W2r.T, b2l.reshape(1, -1), True)
    agg3 = _sc_agg(h2, src, dst2, zrow)

    wl_t = jnp.concatenate([Wml, Wsl], axis=0).T
    wr_t = jnp.concatenate([Wmr, Wsr], axis=0).T
    bc = jnp.concatenate([bml, bsl], axis=0).reshape(1, -1)
    out = _tc_layer(agg3, cnt2, h2, wl_t, wr_t, bc, False)
    return out[:, :DOUT], out[:, DOUT:]


# speedup vs baseline: 1.0026x; 1.0026x over previous
import functools

import jax
import jax.numpy as jnp
from jax import lax
from jax.experimental import pallas as pl
from jax.experimental.pallas import tpu as pltpu
from jax.experimental.pallas import tpu_sc as plsc

N = 10000
E = 320000
D = 128
DOUT = 64

NC = 2
NS = 16
NW = NC * NS
EPW = E // NW
CH = 80
NCHUNK = EPW // CH
NP = 10240
RPT = NP // NS


def _sc_agg_body(h_hbm, src_hbm, dst2_hbm, zrow_hbm, out_hbm,
                 sidx, didx, rows, acc, gsem):
    cid = lax.axis_index("c")
    sid = lax.axis_index("s")
    wid = cid * NS + sid
    r0 = sid * RPT

    pltpu.sync_copy(zrow_hbm, acc.at[pl.ds(r0, RPT)])
    pltpu.sync_copy(src_hbm.at[pl.ds(wid * EPW, EPW)], sidx)
    pltpu.sync_copy(dst2_hbm.at[wid], didx)
    plsc.subcore_barrier()

    pltpu.async_copy(h_hbm.at[sidx.at[pl.ds(0, CH)]], rows.at[0], gsem)

    def pair(j2, carry):
        for b in range(2):
            j = 2 * j2 + b
            pltpu.make_async_copy(h_hbm.at[pl.ds(0, CH)], rows.at[b], gsem).wait()

            @pl.when(j < NCHUNK - 1)
            def _():
                off = (j + 1) * CH
                pltpu.async_copy(h_hbm.at[sidx.at[pl.ds(off, CH)]],
                                 rows.at[1 - b], gsem)

            pltpu.sync_copy(rows.at[b], acc.at[didx.at[j]], add=True)
        return carry

    lax.fori_loop(0, NCHUNK // 2, pair, 0)
    pltpu.make_async_copy(h_hbm.at[pl.ds(0, CH)], rows.at[0], gsem).wait()
    pltpu.sync_copy(rows.at[0], acc.at[didx.at[NCHUNK - 1]], add=True)
    plsc.subcore_barrier()

    pltpu.sync_copy(acc.at[pl.ds(r0, RPT)], out_hbm.at[cid, pl.ds(r0, RPT)])


_sc_agg = pl.kernel(
    _sc_agg_body,
    out_type=jax.ShapeDtypeStruct((NC, NP, D), jnp.float32),
    mesh=plsc.VectorSubcoreMesh(core_axis_name="c", subcore_axis_name="s"),
    scratch_types=[
        pltpu.VMEM((EPW,), jnp.int32),
        pltpu.VMEM((NCHUNK, CH), jnp.int32),
        pltpu.VMEM((2, CH, D), jnp.float32),
        pltpu.VMEM_SHARED((NP, D), jnp.float32),
        pltpu.SemaphoreType.DMA,
    ],
    name="sc_seg_agg",
)


def _sc_cnt_body(dst2_hbm, zrow_hbm, out_hbm, didx, rows1, acc):
    cid = lax.axis_index("c")
    sid = lax.axis_index("s")
    wid = cid * NS + sid
    r0 = sid * RPT

    pltpu.sync_copy(zrow_hbm, acc.at[pl.ds(r0, RPT)])
    pltpu.sync_copy(dst2_hbm.at[wid], didx)
    ones16 = jnp.ones((16,), jnp.float32)

    def fill(r, carry):
        for k in range(D // 16):
            rows1[r, pl.ds(16 * k, 16)] = ones16
        return carry

    lax.fori_loop(0, CH, fill, 0)
    plsc.subcore_barrier()

    def chunk(j, carry):
        pltpu.sync_copy(rows1, acc.at[didx.at[j]], add=True)
        return carry

    lax.fori_loop(0, NCHUNK, chunk, 0)
    plsc.subcore_barrier()

    pltpu.sync_copy(acc.at[pl.ds(r0, RPT)], out_hbm.at[cid, pl.ds(r0, RPT)])


_sc_cnt = pl.kernel(
    _sc_cnt_body,
    out_type=jax.ShapeDtypeStruct((NC, NP, D), jnp.float32),
    mesh=plsc.VectorSubcoreMesh(core_axis_name="c", subcore_axis_name="s"),
    scratch_types=[
        pltpu.VMEM((NCHUNK, CH), jnp.int32),
        pltpu.VMEM((CH, D), jnp.float32),
        pltpu.VMEM_SHARED((NP, D), jnp.float32),
    ],
    name="sc_seg_cnt",
)


_TCB = 1000


def _tc_layer_body(relu, part_ref, cnt_ref, h_ref, wl_ref, wr_ref, b_ref, o_ref):
    p = part_ref[...]
    c = cnt_ref[...]
    rc = 1.0 / jnp.maximum(c[:, 0:1] + c[:, 1:2], 1.0)
    mean = (p[0] + p[1]) * rc
    y = jnp.dot(mean, wl_ref[...], preferred_element_type=jnp.float32)
    y = y + jnp.dot(h_ref[...], wr_ref[...], preferred_element_type=jnp.float32)
    y = y + b_ref[...]
    if relu:
        y = jnp.maximum(y, 0.0)
    o_ref[...] = y


def _tc_layer(part, cnt2, hprev, wl_t, wr_t, b, relu):
    dout = wl_t.shape[1]
    grid = (N // _TCB,)
    return pl.pallas_call(
        functools.partial(_tc_layer_body, relu),
        grid=grid,
        in_specs=[
            pl.BlockSpec((2, _TCB, D), lambda i: (0, i, 0)),
            pl.BlockSpec((_TCB, 2), lambda i: (i, 0)),
            pl.BlockSpec((_TCB, D), lambda i: (i, 0)),
            pl.BlockSpec((D, dout), lambda i: (0, 0)),
            pl.BlockSpec((D, dout), lambda i: (0, 0)),
            pl.BlockSpec((1, dout), lambda i: (0, 0)),
        ],
        out_specs=pl.BlockSpec((_TCB, dout), lambda i: (i, 0)),
        out_shape=jax.ShapeDtypeStruct((N, dout), jnp.float32),
        name="tc_sage_linear",
    )(part, cnt2, hprev, wl_t, wr_t, b)


def kernel(x, edge_index, W1l, b1l, W1r, W2l, b2l, W2r,
           Wml, bml, Wmr, Wsl, bsl, Wsr):
    src = edge_index[0]
    dst = edge_index[1]
    zrow = jnp.zeros((RPT, D), jnp.float32)
    dst2 = dst.reshape(NW, NCHUNK, CH)
    cnt = _sc_cnt(dst2, zrow)
    cnt2 = jnp.concatenate([cnt[0, :, :1], cnt[1, :, :1]], axis=1)
    agg1 = _sc_agg(x, src, dst2, zrow)

    h1 = _tc_layer(agg1, cnt2, x, W1l.T, W1r.T, b1l.reshape(1, -1), True)
    agg2 = _sc_agg(h1, src, dst2, zrow)
    h2 = _tc_layer(agg2, cnt2, h1, W2l.T, W2r.T, b2l.reshape(1, -1), True)
    agg3 = _sc_agg(h2, src, dst2, zrow)

    wl_t = jnp.concatenate([Wml, Wsl], axis=0).T
    wr_t = jnp.concatenate([Wmr, Wsr], axis=0).T
    bc = jnp.concatenate([bml, bsl], axis=0).reshape(1, -1)
    out = _tc_layer(agg3, cnt2, h2, wl_t, wr_t, bc, False)
    return out[:, :DOUT], out[:, DOUT:]
